# baseline (speedup 1.0000x reference)
import functools

import jax
import jax.numpy as jnp
from jax.experimental import pallas as pl
from jax.experimental.pallas import tpu as pltpu

MAX_EDGES = 1600000
MAX_DELAY = 20
MIN_DELAY = 1
TRACE_DECAY = 0.95

_LANES = 128
_BROWS = 500
_GRID = MAX_EDGES // (_BROWS * _LANES)


def _tc_body(ptr_ref, dc_ref, sb_ref, pre_ref, post_ref, pspk_ref,
             out_d_ref, out_s_ref):
    dc = dc_ref[0]
    d = jnp.clip(jnp.round(dc).astype(jnp.int32), MIN_DELAY, MAX_DELAY - 1)
    t = ptr_ref[0] - d
    r = jnp.where(t < 0, t + MAX_DELAY, t)
    delayed = jnp.zeros_like(dc)
    for k in range(MAX_DELAY):
        delayed = jnp.where(r == k, sb_ref[k, 0, :, :], delayed)
    pspk = pspk_ref[0]
    new_pre = pre_ref[0] * TRACE_DECAY + delayed
    new_post = post_ref[0] * TRACE_DECAY + pspk
    out_d_ref[0] = delayed
    out_s_ref[0] = delayed * new_post - pspk * new_pre


@jax.jit
def _run_tc(dc, sb, pre, post, pspk, ptr_mod):
    spec1 = pl.BlockSpec((1, _BROWS, _LANES), lambda i: (i, 0, 0))
    return pl.pallas_call(
        _tc_body,
        grid=(_GRID,),
        in_specs=[
            pl.BlockSpec(memory_space=pltpu.SMEM),
            spec1,
            pl.BlockSpec((MAX_DELAY, 1, _BROWS, _LANES),
                         lambda i: (0, i, 0, 0)),
            spec1,
            spec1,
            spec1,
        ],
        out_specs=[spec1, spec1],
        out_shape=[
            jax.ShapeDtypeStruct((_GRID, _BROWS, _LANES), jnp.float32),
            jax.ShapeDtypeStruct((_GRID, _BROWS, _LANES), jnp.float32),
        ],
    )(ptr_mod, dc, sb, pre, post, pspk)


def kernel(pre_spk, post_spk, delay_continuous, spike_buffer,
           pre_trace_delayed, post_trace, buffer_ptr):
    del pre_spk
    ptr_mod = (buffer_ptr % MAX_DELAY).astype(jnp.int32).reshape((1,))
    r2 = lambda x: x.reshape(_GRID, _BROWS, _LANES)
    delayed, stdp = _run_tc(r2(delay_continuous),
                            spike_buffer.reshape(MAX_DELAY, _GRID, _BROWS, _LANES),
                            r2(pre_trace_delayed), r2(post_trace),
                            r2(post_spk), ptr_mod)
    return (delayed.reshape(MAX_EDGES), stdp.reshape(MAX_EDGES))
